# Initial kernel scaffold; baseline (speedup 1.0000x reference)
import jax
import jax.numpy as jnp
from jax.experimental import pallas as pl


def kernel(nodes, edges, senders, receivers, eW1, eb1, eW2, eb2, nW1, nb1, nW2, nb2):
    raise NotImplementedError("write your pallas kernel here")


# speedup vs baseline: 2.8246x; 2.8246x over previous
import functools

import jax
import jax.numpy as jnp
from jax import lax
from jax.experimental import pallas as pl
from jax.experimental.pallas import tpu as pltpu
from jax.experimental.pallas import tpu_sc as plsc

_NC, _NS = 2, 16
_NW = _NC * _NS
_C = 80
_LANES = 16


def _f32(x):
    return x.astype(jnp.float32)


def _node_projections(nodes2d, eW1r, eW1s):
    n, d = nodes2d.shape

    def body(x_ref, wr_ref, ws_ref, pr_ref, ps_ref):
        x = x_ref[...]
        pr_ref[...] = jnp.dot(x, wr_ref[...], preferred_element_type=jnp.float32)
        ps_ref[...] = jnp.dot(x, ws_ref[...], preferred_element_type=jnp.float32)

    return pl.pallas_call(
        body,
        out_shape=(
            jax.ShapeDtypeStruct((n, d), jnp.float32),
            jax.ShapeDtypeStruct((n, d), jnp.float32),
        ),
    )(nodes2d, eW1r, eW1s)


def _gather_messages(PR, PS, senders, receivers):
    n, d = PR.shape
    e = senders.shape[0]
    epw = e // _NW
    nch = epw // _C
    assert epw * _NW == e and nch * _C == epw

    mesh = plsc.VectorSubcoreMesh(core_axis_name="c", subcore_axis_name="s")

    @functools.partial(
        pl.kernel,
        out_type=jax.ShapeDtypeStruct((e, d), jnp.float32),
        mesh=mesh,
        scratch_types=[
            pltpu.VMEM((_C,), jnp.int32),
            pltpu.VMEM((_C,), jnp.int32),
            pltpu.VMEM((_C, d), jnp.float32),
            pltpu.VMEM((_C, d), jnp.float32),
            pltpu.VMEM((_C, d), jnp.float32),
            pltpu.SemaphoreType.DMA,
            pltpu.SemaphoreType.DMA,
        ],
    )
    def k(pr_hbm, ps_hbm, snd_hbm, rcv_hbm, g_hbm, ridx, sidx, bufr, bufs, bufo, semr, sems):
        wid = lax.axis_index("s") * _NC + lax.axis_index("c")
        base_w = wid * epw

        def chunk(j, carry):
            base = base_w + j * _C
            pltpu.sync_copy(rcv_hbm.at[pl.ds(base, _C)], ridx)
            pltpu.sync_copy(snd_hbm.at[pl.ds(base, _C)], sidx)
            cr = pltpu.async_copy(pr_hbm.at[ridx], bufr, semr)
            cs = pltpu.async_copy(ps_hbm.at[sidx], bufs, sems)
            cr.wait()
            cs.wait()

            def row(r, c):
                for v in range(d // _LANES):
                    sl = pl.ds(v * _LANES, _LANES)
                    bufo[r, sl] = bufr[r, sl] + bufs[r, sl]
                return c

            lax.fori_loop(0, _C, row, 0)
            pltpu.sync_copy(bufo, g_hbm.at[pl.ds(base, _C)])
            return carry

        lax.fori_loop(0, nch, chunk, 0)

    return k(PR, PS, senders, receivers)


def _edge_mlp(edges2d, G, eW1a, eb1, eW2, eb2):
    e, d = edges2d.shape
    eb = 4000
    assert e % eb == 0

    def body(x_ref, g_ref, w1_ref, b1_ref, w2_ref, b2_ref, o_ref):
        h = jnp.dot(x_ref[...], w1_ref[...], preferred_element_type=jnp.float32)
        h = jnp.maximum(h + g_ref[...] + b1_ref[...], 0.0)
        o_ref[...] = (
            jnp.dot(h, w2_ref[...], preferred_element_type=jnp.float32) + b2_ref[...]
        )

    return pl.pallas_call(
        body,
        grid=(e // eb,),
        in_specs=[
            pl.BlockSpec((eb, d), lambda i: (i, 0)),
            pl.BlockSpec((eb, d), lambda i: (i, 0)),
            pl.BlockSpec((d, d), lambda i: (0, 0)),
            pl.BlockSpec((1, d), lambda i: (0, 0)),
            pl.BlockSpec((d, d), lambda i: (0, 0)),
            pl.BlockSpec((1, d), lambda i: (0, 0)),
        ],
        out_specs=pl.BlockSpec((eb, d), lambda i: (i, 0)),
        out_shape=jax.ShapeDtypeStruct((e, d), jnp.float32),
    )(edges2d, G, eW1a, eb1, eW2, eb2)


def _scatter_inbox(edges_new, receivers, zeros_nd):
    e, d = edges_new.shape
    n = zeros_nd.shape[0]
    epw = e // _NW
    nch = epw // _C
    stripe = (n // (8 * _NS)) * 8
    rem = n - stripe * _NS

    mesh = plsc.VectorSubcoreMesh(core_axis_name="c", subcore_axis_name="s")

    @functools.partial(
        pl.kernel,
        out_type=jax.ShapeDtypeStruct((_NC, n, d), jnp.float32),
        mesh=mesh,
        scratch_types=[
            pltpu.VMEM((_C,), jnp.int32),
            pltpu.VMEM((_C, d), jnp.float32),
            pltpu.VMEM_SHARED((n, d), jnp.float32),
        ],
    )
    def k(en_hbm, rcv_hbm, z_hbm, out_hbm, ridx, rows, acc):
        cid = lax.axis_index("c")
        sid = lax.axis_index("s")
        wid = cid * _NS + sid
        base_w = wid * epw

        row0 = sid * stripe
        pltpu.sync_copy(z_hbm.at[pl.ds(row0, stripe)], acc.at[pl.ds(row0, stripe)])
        if rem:
            @pl.when(sid == _NS - 1)
            def _():
                r0 = stripe * _NS
                pltpu.sync_copy(z_hbm.at[pl.ds(r0, rem)], acc.at[pl.ds(r0, rem)])
        plsc.subcore_barrier()

        def chunk(j, carry):
            base = base_w + j * _C
            pltpu.sync_copy(rcv_hbm.at[pl.ds(base, _C)], ridx)
            pltpu.sync_copy(en_hbm.at[pl.ds(base, _C)], rows)
            pltpu.sync_copy(rows, acc.at[ridx], add=True)
            return carry

        lax.fori_loop(0, nch, chunk, 0)
        plsc.subcore_barrier()
        pltpu.sync_copy(acc.at[pl.ds(row0, stripe)], out_hbm.at[cid, pl.ds(row0, stripe)])
        if rem:
            @pl.when(sid == _NS - 1)
            def _():
                r0 = stripe * _NS
                pltpu.sync_copy(acc.at[pl.ds(r0, rem)], out_hbm.at[cid, pl.ds(r0, rem)])

    return k(edges_new, receivers, zeros_nd)


def _node_mlp(nodes2d, p0, p1, nW1a, nW1b, nb1, nW2, nb2):
    n, d = nodes2d.shape

    def body(x_ref, p0_ref, p1_ref, wa_ref, wb_ref, b1_ref, w2_ref, b2_ref, o_ref):
        x = x_ref[...]
        inbox = p0_ref[...] + p1_ref[...]
        h = (
            jnp.dot(x, wa_ref[...], preferred_element_type=jnp.float32)
            + jnp.dot(inbox, wb_ref[...], preferred_element_type=jnp.float32)
            + b1_ref[...]
        )
        h = jnp.maximum(h, 0.0)
        o_ref[...] = (
            x + jnp.dot(h, w2_ref[...], preferred_element_type=jnp.float32) + b2_ref[...]
        )

    return pl.pallas_call(
        body,
        out_shape=jax.ShapeDtypeStruct((n, d), jnp.float32),
    )(nodes2d, p0, p1, nW1a, nW1b, nb1, nW2, nb2)


def kernel(nodes, edges, senders, receivers, eW1, eb1, eW2, eb2, nW1, nb1, nW2, nb2):
    b, n, d = nodes.shape
    e = edges.shape[1]
    nodes2d = nodes.reshape(n, d)
    edges2d = edges.reshape(e, d)

    eW1a, eW1r, eW1s = eW1[:d], eW1[d : 2 * d], eW1[2 * d :]
    nW1a, nW1b = nW1[:d], nW1[d:]
    eb1_2 = eb1.reshape(1, d)
    eb2_2 = eb2.reshape(1, d)
    nb1_2 = nb1.reshape(1, d)
    nb2_2 = nb2.reshape(1, d)

    PR, PS = _node_projections(nodes2d, eW1r, eW1s)
    G = _gather_messages(PR, PS, senders, receivers)
    edges_new = _edge_mlp(edges2d, G, eW1a, eb1_2, eW2, eb2_2)
    zeros_nd = jnp.zeros((n, d), jnp.float32)
    partials = _scatter_inbox(edges_new, receivers, zeros_nd)
    nodes_out = _node_mlp(
        nodes2d, partials[0], partials[1], nW1a, nW1b, nb1_2, nW2, nb2_2
    )
    return nodes_out.reshape(b, n, d), edges_new.reshape(b, e, d)


# speedup vs baseline: 3.2308x; 1.1438x over previous
import functools

import jax
import jax.numpy as jnp
from jax import lax
from jax.experimental import pallas as pl
from jax.experimental.pallas import tpu as pltpu
from jax.experimental.pallas import tpu_sc as plsc

_NC, _NS = 2, 16
_NW = _NC * _NS
_C = 40
_CS = 40
_NBUF = 5
_LANES = 16


def _node_projections(nodes2d, eW1r, eW1s):
    n, d = nodes2d.shape

    def body(x_ref, wr_ref, ws_ref, pr_ref, ps_ref):
        x = x_ref[...]
        pr_ref[...] = jnp.dot(x, wr_ref[...], preferred_element_type=jnp.float32)
        ps_ref[...] = jnp.dot(x, ws_ref[...], preferred_element_type=jnp.float32)

    return pl.pallas_call(
        body,
        out_shape=(
            jax.ShapeDtypeStruct((n, d), jnp.float32),
            jax.ShapeDtypeStruct((n, d), jnp.float32),
        ),
    )(nodes2d, eW1r, eW1s)


def _gather_messages(PR, PS, snd3, rcv3):
    n, d = PR.shape
    nw, nch, c = rcv3.shape
    e = nw * nch * c
    epw = nch * c
    assert nch % _NBUF == 0

    mesh = plsc.VectorSubcoreMesh(core_axis_name="c", subcore_axis_name="s")

    @functools.partial(
        pl.kernel,
        out_type=jax.ShapeDtypeStruct((e, d), jnp.float32),
        mesh=mesh,
        scratch_types=(
            [pltpu.VMEM((nch, c), jnp.int32)] * 2
            + [pltpu.VMEM((c, d), jnp.float32)] * (2 * _NBUF)
            + [pltpu.SemaphoreType.DMA] * (3 * _NBUF)
        ),
    )
    def k(pr_hbm, ps_hbm, snd_hbm, rcv_hbm, g_hbm, *scr):
        ridx2, sidx2 = scr[0], scr[1]
        bufr = scr[2 : 2 + _NBUF]
        bufs = scr[2 + _NBUF : 2 + 2 * _NBUF]
        semr = scr[2 + 2 * _NBUF : 2 + 3 * _NBUF]
        sems = scr[2 + 3 * _NBUF : 2 + 4 * _NBUF]
        semo = scr[2 + 4 * _NBUF : 2 + 5 * _NBUF]

        wid = lax.axis_index("s") * _NC + lax.axis_index("c")
        base_w = wid * epw

        pltpu.sync_copy(rcv_hbm.at[wid], ridx2)
        pltpu.sync_copy(snd_hbm.at[wid], sidx2)

        for b in range(_NBUF):
            pltpu.async_copy(pr_hbm.at[ridx2.at[b]], bufr[b], semr[b])
            pltpu.async_copy(ps_hbm.at[sidx2.at[b]], bufs[b], sems[b])

        def body(j5, carry):
            jj = j5 * _NBUF
            for b in range(_NBUF):
                j = jj + b
                pltpu.make_async_copy(pr_hbm.at[ridx2.at[j]], bufr[b], semr[b]).wait()
                pltpu.make_async_copy(ps_hbm.at[sidx2.at[j]], bufs[b], sems[b]).wait()
                br, bs = bufr[b], bufs[b]

                def row(r, cc):
                    for v in range(d // _LANES):
                        sl = pl.ds(v * _LANES, _LANES)
                        br[r, sl] = br[r, sl] + bs[r, sl]
                    return cc

                lax.fori_loop(0, c, row, 0, unroll=2)
                pltpu.async_copy(br, g_hbm.at[pl.ds(base_w + j * c, c)], semo[b])
            for b in range(_NBUF):
                j = jj + b
                pltpu.make_async_copy(
                    bufr[b], g_hbm.at[pl.ds(base_w + j * c, c)], semo[b]
                ).wait()
                nj = jj + _NBUF + b

                @pl.when(nj < nch)
                def _():
                    pltpu.async_copy(pr_hbm.at[ridx2.at[nj]], bufr[b], semr[b])
                    pltpu.async_copy(ps_hbm.at[sidx2.at[nj]], bufs[b], sems[b])

            return carry

        lax.fori_loop(0, nch // _NBUF, body, 0)

    return k(PR, PS, snd3, rcv3)


def _edge_mlp(edges2d, G, eW1a, eb1, eW2, eb2):
    e, d = edges2d.shape
    eb = 4000
    assert e % eb == 0

    def body(x_ref, g_ref, w1_ref, b1_ref, w2_ref, b2_ref, o_ref):
        h = jnp.dot(x_ref[...], w1_ref[...], preferred_element_type=jnp.float32)
        h = jnp.maximum(h + g_ref[...] + b1_ref[...], 0.0)
        o_ref[...] = (
            jnp.dot(h, w2_ref[...], preferred_element_type=jnp.float32) + b2_ref[...]
        )

    return pl.pallas_call(
        body,
        grid=(e // eb,),
        in_specs=[
            pl.BlockSpec((eb, d), lambda i: (i, 0)),
            pl.BlockSpec((eb, d), lambda i: (i, 0)),
            pl.BlockSpec((d, d), lambda i: (0, 0)),
            pl.BlockSpec((1, d), lambda i: (0, 0)),
            pl.BlockSpec((d, d), lambda i: (0, 0)),
            pl.BlockSpec((1, d), lambda i: (0, 0)),
        ],
        out_specs=pl.BlockSpec((eb, d), lambda i: (i, 0)),
        out_shape=jax.ShapeDtypeStruct((e, d), jnp.float32),
    )(edges2d, G, eW1a, eb1, eW2, eb2)


def _scatter_inbox(edges_new, rcv3, zeros_nd):
    e, d = edges_new.shape
    n = zeros_nd.shape[0]
    nw, nch, c = rcv3.shape
    epw = nch * c
    assert nch % _NBUF == 0
    stripe = (n // (8 * _NS)) * 8
    rem = n - stripe * _NS

    mesh = plsc.VectorSubcoreMesh(core_axis_name="c", subcore_axis_name="s")

    @functools.partial(
        pl.kernel,
        out_type=jax.ShapeDtypeStruct((_NC, n, d), jnp.float32),
        mesh=mesh,
        scratch_types=(
            [pltpu.VMEM((c,), jnp.int32)] * _NBUF
            + [pltpu.VMEM((c, d), jnp.float32)] * _NBUF
            + [pltpu.VMEM_SHARED((n, d), jnp.float32)]
            + [pltpu.SemaphoreType.DMA] * (3 * _NBUF)
        ),
    )
    def k(en_hbm, rcv_hbm, z_hbm, out_hbm, *scr):
        idxb = scr[0:_NBUF]
        rows = scr[_NBUF : 2 * _NBUF]
        acc = scr[2 * _NBUF]
        semi = scr[2 * _NBUF + 1 : 3 * _NBUF + 1]
        seme = scr[3 * _NBUF + 1 : 4 * _NBUF + 1]
        sema = scr[4 * _NBUF + 1 : 5 * _NBUF + 1]

        cid = lax.axis_index("c")
        sid = lax.axis_index("s")
        wid = cid * _NS + sid
        base_w = wid * epw

        row0 = sid * stripe
        pltpu.sync_copy(z_hbm.at[pl.ds(row0, stripe)], acc.at[pl.ds(row0, stripe)])
        if rem:
            @pl.when(sid == _NS - 1)
            def _():
                r0 = stripe * _NS
                pltpu.sync_copy(z_hbm.at[pl.ds(r0, rem)], acc.at[pl.ds(r0, rem)])
        plsc.subcore_barrier()

        for b in range(_NBUF):
            pltpu.async_copy(rcv_hbm.at[wid, b], idxb[b], semi[b])
            pltpu.async_copy(en_hbm.at[pl.ds(base_w + b * c, c)], rows[b], seme[b])

        def body(j5, carry):
            jj = j5 * _NBUF
            scats = []
            for b in range(_NBUF):
                j = jj + b
                pltpu.make_async_copy(rcv_hbm.at[wid, j], idxb[b], semi[b]).wait()
                pltpu.make_async_copy(
                    en_hbm.at[pl.ds(base_w + j * c, c)], rows[b], seme[b]
                ).wait()
                scats.append(
                    pltpu.async_copy(rows[b], acc.at[idxb[b]], sema[b], add=True)
                )
            for b in range(_NBUF):
                scats[b].wait()
                nj = jj + _NBUF + b

                @pl.when(nj < nch)
                def _():
                    pltpu.async_copy(rcv_hbm.at[wid, nj], idxb[b], semi[b])
                    pltpu.async_copy(
                        en_hbm.at[pl.ds(base_w + nj * c, c)], rows[b], seme[b]
                    )

            return carry

        lax.fori_loop(0, nch // _NBUF, body, 0)

        plsc.subcore_barrier()
        pltpu.sync_copy(acc.at[pl.ds(row0, stripe)], out_hbm.at[cid, pl.ds(row0, stripe)])
        if rem:
            @pl.when(sid == _NS - 1)
            def _():
                r0 = stripe * _NS
                pltpu.sync_copy(acc.at[pl.ds(r0, rem)], out_hbm.at[cid, pl.ds(r0, rem)])

    return k(edges_new, rcv3, zeros_nd)


def _node_mlp(nodes2d, p0, p1, nW1a, nW1b, nb1, nW2, nb2):
    n, d = nodes2d.shape

    def body(x_ref, p0_ref, p1_ref, wa_ref, wb_ref, b1_ref, w2_ref, b2_ref, o_ref):
        x = x_ref[...]
        inbox = p0_ref[...] + p1_ref[...]
        h = (
            jnp.dot(x, wa_ref[...], preferred_element_type=jnp.float32)
            + jnp.dot(inbox, wb_ref[...], preferred_element_type=jnp.float32)
            + b1_ref[...]
        )
        h = jnp.maximum(h, 0.0)
        o_ref[...] = (
            x + jnp.dot(h, w2_ref[...], preferred_element_type=jnp.float32) + b2_ref[...]
        )

    return pl.pallas_call(
        body,
        out_shape=jax.ShapeDtypeStruct((n, d), jnp.float32),
    )(nodes2d, p0, p1, nW1a, nW1b, nb1, nW2, nb2)


def kernel(nodes, edges, senders, receivers, eW1, eb1, eW2, eb2, nW1, nb1, nW2, nb2):
    b, n, d = nodes.shape
    e = edges.shape[1]
    nodes2d = nodes.reshape(n, d)
    edges2d = edges.reshape(e, d)
    epw = e // _NW
    nch = epw // _C
    assert epw * _NW == e and nch * _C == epw and nch % _NBUF == 0
    snd3 = senders.reshape(_NW, nch, _C)
    rcv3 = receivers.reshape(_NW, nch, _C)
    rcv3s = receivers.reshape(_NW, epw // _CS, _CS)

    eW1a, eW1r, eW1s = eW1[:d], eW1[d : 2 * d], eW1[2 * d :]
    nW1a, nW1b = nW1[:d], nW1[d:]
    eb1_2 = eb1.reshape(1, d)
    eb2_2 = eb2.reshape(1, d)
    nb1_2 = nb1.reshape(1, d)
    nb2_2 = nb2.reshape(1, d)

    PR, PS = _node_projections(nodes2d, eW1r, eW1s)
    G = _gather_messages(PR, PS, snd3, rcv3)
    edges_new = _edge_mlp(edges2d, G, eW1a, eb1_2, eW2, eb2_2)
    zeros_nd = jnp.zeros((n, d), jnp.float32)
    partials = _scatter_inbox(edges_new, rcv3s, zeros_nd)
    nodes_out = _node_mlp(
        nodes2d, partials[0], partials[1], nW1a, nW1b, nb1_2, nW2, nb2_2
    )
    return nodes_out.reshape(b, n, d), edges_new.reshape(b, e, d)
